# baseline (speedup 1.0000x reference)
import functools

import jax
import jax.numpy as jnp
from jax import lax
from jax.experimental import pallas as pl
from jax.experimental.pallas import tpu as pltpu
from jax.experimental.pallas import tpu_sc as plsc

B = 32
NOISE_STD = 0.05


def _coef_kernel(t_epl_hbm, idx_hbm, atab_hbm, btab_hbm,
                 t_out, a_out, b_out,
                 idx_v, t_v, a_v, b_v, sem_a, sem_b):
    cid = lax.axis_index("c")
    sid = lax.axis_index("s")

    @pl.when(jnp.logical_and(cid == 0, sid == 0))
    def _():
        pltpu.sync_copy(idx_hbm, idx_v)
        pltpu.async_copy(t_epl_hbm.at[idx_v], t_v, sem_a).wait()
        cp_a = pltpu.async_copy(atab_hbm.at[t_v], a_v, sem_a)
        cp_b = pltpu.async_copy(btab_hbm.at[t_v], b_v, sem_b)
        cp_a.wait()
        cp_b.wait()
        pltpu.sync_copy(t_v, t_out)
        pltpu.sync_copy(a_v, a_out)
        pltpu.sync_copy(b_v, b_out)


def _gather_coefs(t_epl, random_indices, atab, btab):
    mesh = plsc.VectorSubcoreMesh(core_axis_name="c", subcore_axis_name="s",
                                  num_cores=1)
    kern = functools.partial(
        pl.kernel,
        mesh=mesh,
        out_type=[
            jax.ShapeDtypeStruct((B,), jnp.int32),
            jax.ShapeDtypeStruct((B,), jnp.float32),
            jax.ShapeDtypeStruct((B,), jnp.float32),
        ],
        scratch_types=[
            pltpu.VMEM((B,), jnp.int32),
            pltpu.VMEM((B,), jnp.int32),
            pltpu.VMEM((B,), jnp.float32),
            pltpu.VMEM((B,), jnp.float32),
            pltpu.SemaphoreType.DMA,
            pltpu.SemaphoreType.DMA,
        ],
    )(_coef_kernel)
    return kern(t_epl, random_indices, atab, btab)


def _combine_kernel(a_ref, b_ref, x_ref, n_ref, o_ref):
    i = pl.program_id(0)
    o_ref[...] = a_ref[i] * x_ref[...] + (b_ref[i] * NOISE_STD) * n_ref[...]


def _combine(a, b, x, n):
    _, C, H, W = x.shape
    return pl.pallas_call(
        _combine_kernel,
        grid=(B,),
        in_specs=[
            pl.BlockSpec(memory_space=pltpu.SMEM),
            pl.BlockSpec(memory_space=pltpu.SMEM),
            pl.BlockSpec((1, C, H, W), lambda i: (i, 0, 0, 0)),
            pl.BlockSpec((1, C, H, W), lambda i: (i, 0, 0, 0)),
        ],
        out_specs=pl.BlockSpec((1, C, H, W), lambda i: (i, 0, 0, 0)),
        out_shape=jax.ShapeDtypeStruct(x.shape, jnp.float32),
    )(a, b, x, n)


def kernel(x_0, alphas_bar_sqrt, one_minus_alphas_bar_sqrt, t_epl, random_indices, noise):
    t, a, b = _gather_coefs(t_epl, random_indices,
                            alphas_bar_sqrt, one_minus_alphas_bar_sqrt)
    out = _combine(a, b, x_0, noise)
    return (out, t.reshape(-1, 1))


# speedup vs baseline: 1.0517x; 1.0517x over previous
import functools

import jax
import jax.numpy as jnp
from jax import lax
from jax.experimental import pallas as pl
from jax.experimental.pallas import tpu as pltpu
from jax.experimental.pallas import tpu_sc as plsc

B = 32
NOISE_STD = 0.05


def _t_gather_kernel(t_epl_hbm, idx_hbm, t_out, idx_v, t_v, sem):
    cid = lax.axis_index("c")
    sid = lax.axis_index("s")

    @pl.when(jnp.logical_and(cid == 0, sid == 0))
    def _():
        pltpu.sync_copy(idx_hbm, idx_v)
        pltpu.async_copy(t_epl_hbm.at[idx_v], t_v, sem).wait()
        pltpu.sync_copy(t_v, t_out)


def _gather_t(t_epl, random_indices):
    mesh = plsc.VectorSubcoreMesh(core_axis_name="c", subcore_axis_name="s",
                                  num_cores=1)
    kern = functools.partial(
        pl.kernel,
        mesh=mesh,
        out_type=jax.ShapeDtypeStruct((B,), jnp.int32),
        scratch_types=[
            pltpu.VMEM((B,), jnp.int32),
            pltpu.VMEM((B,), jnp.int32),
            pltpu.SemaphoreType.DMA,
        ],
    )(_t_gather_kernel)
    return kern(t_epl, random_indices)


def _combine_kernel(idx_ref, t_epl_ref, atab_ref, btab_ref, x_ref, n_ref, o_ref):
    i = pl.program_id(0)
    t = t_epl_ref[idx_ref[i]]
    a = atab_ref[t]
    b = btab_ref[t] * NOISE_STD
    o_ref[...] = a * x_ref[...] + b * n_ref[...]


def _combine(idx, t_epl, atab, btab, x, n):
    _, C, H, W = x.shape
    return pl.pallas_call(
        _combine_kernel,
        grid=(B,),
        in_specs=[
            pl.BlockSpec(memory_space=pltpu.SMEM),
            pl.BlockSpec(memory_space=pltpu.SMEM),
            pl.BlockSpec(memory_space=pltpu.SMEM),
            pl.BlockSpec(memory_space=pltpu.SMEM),
            pl.BlockSpec((1, C, H, W), lambda i: (i, 0, 0, 0)),
            pl.BlockSpec((1, C, H, W), lambda i: (i, 0, 0, 0)),
        ],
        out_specs=pl.BlockSpec((1, C, H, W), lambda i: (i, 0, 0, 0)),
        out_shape=jax.ShapeDtypeStruct(x.shape, jnp.float32),
    )(idx, t_epl, atab, btab, x, n)


def kernel(x_0, alphas_bar_sqrt, one_minus_alphas_bar_sqrt, t_epl, random_indices, noise):
    t = _gather_t(t_epl, random_indices)
    out = _combine(random_indices, t_epl, alphas_bar_sqrt,
                   one_minus_alphas_bar_sqrt, x_0, noise)
    return (out, t.reshape(-1, 1))
